# baseline (speedup 1.0000x reference)
import functools
import math

import jax
import jax.numpy as jnp
from jax import lax
from jax.experimental import pallas as pl
from jax.experimental.pallas import tpu as pltpu
from jax.experimental.pallas import tpu_sc as plsc

_VOCAB = 100000
_EMB = 64
_B = 4096
_L = 200
_SCALE = math.sqrt(_EMB)

_NC = 2
_NS = 16
_NW = _NC * _NS

_B_TOTAL = _B * _L
_B_PER_W = _B_TOTAL // _NW
_CHUNK = 400
_N_CHUNKS = _B_PER_W // _CHUNK
_NBUF = 4


_RCHUNK = 400
_N_RCHUNKS_TOT = _VOCAB // _RCHUNK


@functools.partial(
    pl.kernel,
    out_type=jax.ShapeDtypeStruct((_VOCAB * _EMB,), jnp.float32),
    mesh=plsc.VectorSubcoreMesh(core_axis_name="c", subcore_axis_name="s"),
    scratch_types=[
        pltpu.VMEM((_RCHUNK, _EMB), jnp.float32),
        pltpu.VMEM((_RCHUNK * _EMB,), jnp.float32),
    ],
    compiler_params=pltpu.CompilerParams(use_tc_tiling_on_sc=True),
)
def _sc_scale(table_hbm, out_hbm, in_v, out_v):
    wid = lax.axis_index("s") * _NC + lax.axis_index("c")
    nk = (_N_RCHUNKS_TOT - wid + _NW - 1) // _NW

    @pl.loop(0, nk)
    def _rchunk_loop(k):
        row = (wid + k * _NW) * _RCHUNK
        pltpu.sync_copy(table_hbm.at[pl.ds(row, _RCHUNK)], in_v)

        @pl.loop(0, _RCHUNK)
        def _row_loop(i):
            for j in range(_EMB // 16):
                v = in_v[i, pl.ds(j * 16, 16)]
                out_v[pl.ds(i * _EMB + j * 16, 16)] = v * _SCALE

        pltpu.sync_copy(
            out_v, out_hbm.at[pl.ds(row * _EMB, _RCHUNK * _EMB)])


_OUT_ROWS = _L * 8 * (_B // 128) * 8


@functools.partial(
    pl.kernel,
    out_type=jax.ShapeDtypeStruct((_OUT_ROWS, 128), jnp.float32),
    mesh=plsc.VectorSubcoreMesh(core_axis_name="c", subcore_axis_name="s"),
    scratch_types=[
        pltpu.VMEM((128, _L), jnp.int32),
        pltpu.VMEM((_L * 128,), jnp.int32),
        pltpu.VMEM((_NBUF, 128, _EMB), jnp.float32),
        pltpu.VMEM((_NBUF, _EMB, 129), jnp.float32),
        pltpu.SemaphoreType.DMA((_NBUF,)),
        pltpu.SemaphoreType.DMA((_NBUF,)),
    ],
    compiler_params=pltpu.CompilerParams(
        use_tc_tiling_on_sc=False, needs_layout_passes=False),
)
def _sc_gather(table_hbm, tok_hbm, out_hbm, idx_v, idx_t, rows, trans,
               sem_g, sem_o):
    wid = lax.axis_index("s") * _NC + lax.axis_index("c")
    iota = lax.iota(jnp.int32, 16)
    zeros = jnp.zeros((16,), jnp.int32)

    pltpu.sync_copy(tok_hbm.at[pl.ds(wid * 128, 128)], idx_v)

    @pl.loop(0, _L)
    def _idx_loop(l):
        for k in range(8):
            v = plsc.load_gather(idx_v, [iota + (k * 16), zeros + l])
            idx_t[pl.ds(l * 128 + k * 16, 16)] = v

    def start_gather(b, l):
        pltpu.async_copy(
            table_hbm.at[idx_t.at[pl.ds(l * 128, 128)]],
            rows.at[b], sem_g.at[b])

    def wait_gather(b):
        pltpu.make_async_copy(
            table_hbm.at[idx_t.at[pl.ds(0, 128)]],
            rows.at[b], sem_g.at[b]).wait()

    def transpose_scale(b):
        @pl.loop(0, 128)
        def _t_loop(bb):
            for c in range(4):
                v = rows[b, bb, pl.ds(c * 16, 16)]
                plsc.store_scatter(
                    trans.at[b], [iota + (c * 16), zeros + bb], v)

    def start_outs(b, l):
        for et in range(8):
            row0 = ((l * 8 + et) * 32 + wid) * 8
            pltpu.async_copy(
                trans.at[b, pl.ds(et * 8, 8), pl.ds(0, 128)],
                out_hbm.at[pl.ds(row0, 8)], sem_o.at[b])

    def wait_outs(b):
        for et in range(8):
            pltpu.make_async_copy(
                trans.at[b, pl.ds(0, 8), pl.ds(0, 128)],
                out_hbm.at[pl.ds(0, 8)], sem_o.at[b]).wait()

    def group(i, first, last):
        for b in range(_NBUF):
            l = i + b
            wait_gather(b)
            if not first:
                wait_outs(b)
            transpose_scale(b)
            start_outs(b, l)
            if not last:
                start_gather(b, l + _NBUF)

    for b in range(_NBUF):
        start_gather(b, b)
    group(0, True, False)

    @pl.loop(1, _L // _NBUF - 1)
    def _main_loop(gi):
        group(gi * _NBUF, False, False)

    group(_L - _NBUF, False, True)
    for b in range(_NBUF):
        wait_outs(b)


def kernel(tokens, word_vectors):
    scaled = _sc_scale(word_vectors).reshape(_VOCAB, _EMB)
    out = _sc_gather(scaled, tokens)
    x5 = out.reshape(_L, 8, _B // 128, 8, 128)
    return x5.transpose(2, 4, 0, 1, 3).reshape(_B, _L, _EMB)


# speedup vs baseline: 1.0341x; 1.0341x over previous
import functools
import math

import jax
import jax.numpy as jnp
from jax import lax
from jax.experimental import pallas as pl
from jax.experimental.pallas import tpu as pltpu
from jax.experimental.pallas import tpu_sc as plsc

_VOCAB = 100000
_EMB = 64
_B = 4096
_L = 200
_SCALE = math.sqrt(_EMB)

_NC = 2
_NS = 16
_NW = _NC * _NS

_B_TOTAL = _B * _L
_B_PER_W = _B_TOTAL // _NW
_CHUNK = 400
_N_CHUNKS = _B_PER_W // _CHUNK
_NBUF = 4


_RCHUNK = 400
_N_RCHUNKS_TOT = _VOCAB // _RCHUNK


@functools.partial(
    pl.kernel,
    out_type=jax.ShapeDtypeStruct((_VOCAB * _EMB,), jnp.float32),
    mesh=plsc.VectorSubcoreMesh(core_axis_name="c", subcore_axis_name="s"),
    scratch_types=[
        pltpu.VMEM((_RCHUNK, _EMB), jnp.float32),
        pltpu.VMEM((_RCHUNK * _EMB,), jnp.float32),
    ],
    compiler_params=pltpu.CompilerParams(use_tc_tiling_on_sc=True),
)
def _sc_scale(table_hbm, out_hbm, in_v, out_v):
    wid = lax.axis_index("s") * _NC + lax.axis_index("c")
    nk = (_N_RCHUNKS_TOT - wid + _NW - 1) // _NW

    @pl.loop(0, nk)
    def _rchunk_loop(k):
        row = (wid + k * _NW) * _RCHUNK
        pltpu.sync_copy(table_hbm.at[pl.ds(row, _RCHUNK)], in_v)

        @pl.loop(0, _RCHUNK)
        def _row_loop(i):
            for j in range(_EMB // 16):
                v = in_v[i, pl.ds(j * 16, 16)]
                out_v[pl.ds(i * _EMB + j * 16, 16)] = v * _SCALE

        pltpu.sync_copy(
            out_v, out_hbm.at[pl.ds(row * _EMB, _RCHUNK * _EMB)])


_OUT_ROWS = _L * 8 * (_B // 128) * 8


@functools.partial(
    pl.kernel,
    out_type=jax.ShapeDtypeStruct((_OUT_ROWS, 128), jnp.float32),
    mesh=plsc.VectorSubcoreMesh(core_axis_name="c", subcore_axis_name="s"),
    scratch_types=[
        pltpu.VMEM((128, _L), jnp.int32),
        pltpu.VMEM((_L * 128,), jnp.int32),
        pltpu.VMEM((_NBUF, 128, _EMB), jnp.float32),
        pltpu.VMEM((_NBUF, _EMB, 129), jnp.float32),
        pltpu.SemaphoreType.DMA((_NBUF,)),
        pltpu.SemaphoreType.DMA((_NBUF,)),
    ],
    compiler_params=pltpu.CompilerParams(
        use_tc_tiling_on_sc=False, needs_layout_passes=False),
)
def _sc_gather(table_hbm, tok_hbm, out_hbm, idx_v, idx_t, rows, trans,
               sem_g, sem_o):
    wid = lax.axis_index("s") * _NC + lax.axis_index("c")
    iota = lax.iota(jnp.int32, 16)
    zeros = jnp.zeros((16,), jnp.int32)

    pltpu.sync_copy(tok_hbm.at[pl.ds(wid * 128, 128)], idx_v)

    @pl.loop(0, _L)
    def _idx_loop(l):
        for k in range(8):
            v = plsc.load_gather(idx_v, [iota + (k * 16), zeros + l])
            idx_t[pl.ds(l * 128 + k * 16, 16)] = v

    def start_gather(b, l):
        pltpu.async_copy(
            table_hbm.at[idx_t.at[pl.ds(l * 128, 128)]],
            rows.at[b], sem_g.at[b])

    def wait_gather(b):
        pltpu.make_async_copy(
            table_hbm.at[idx_t.at[pl.ds(0, 128)]],
            rows.at[b], sem_g.at[b]).wait()

    rowidx = [iota + (c * 16) for c in range(4)]

    def transpose_scale(b):
        @pl.loop(0, 128, unroll=4)
        def _t_loop(bb):
            colv = zeros + bb
            for c in range(4):
                v = rows[b, bb, pl.ds(c * 16, 16)]
                plsc.store_scatter(trans.at[b], [rowidx[c], colv], v)

    def start_outs(b, l):
        for et in range(8):
            row0 = ((l * 8 + et) * 32 + wid) * 8
            pltpu.async_copy(
                trans.at[b, pl.ds(et * 8, 8), pl.ds(0, 128)],
                out_hbm.at[pl.ds(row0, 8)], sem_o.at[b])

    def wait_outs(b):
        for et in range(8):
            pltpu.make_async_copy(
                trans.at[b, pl.ds(0, 8), pl.ds(0, 128)],
                out_hbm.at[pl.ds(0, 8)], sem_o.at[b]).wait()

    def group(i, first, last):
        for b in range(_NBUF):
            l = i + b
            wait_gather(b)
            if not first:
                wait_outs(b)
            transpose_scale(b)
            start_outs(b, l)
            if not last:
                start_gather(b, l + _NBUF)

    for b in range(_NBUF):
        start_gather(b, b)
    group(0, True, False)

    @pl.loop(1, _L // _NBUF - 1)
    def _main_loop(gi):
        group(gi * _NBUF, False, False)

    group(_L - _NBUF, False, True)
    for b in range(_NBUF):
        wait_outs(b)


def kernel(tokens, word_vectors):
    scaled = _sc_scale(word_vectors).reshape(_VOCAB, _EMB)
    out = _sc_gather(scaled, tokens)
    x5 = out.reshape(_L, 8, _B // 128, 8, 128)
    return x5.transpose(2, 4, 0, 1, 3).reshape(_B, _L, _EMB)


# speedup vs baseline: 1.2882x; 1.2457x over previous
import functools
import math

import jax
import jax.numpy as jnp
from jax import lax
from jax.experimental import pallas as pl
from jax.experimental.pallas import tpu as pltpu
from jax.experimental.pallas import tpu_sc as plsc

_VOCAB = 100000
_EMB = 64
_B = 4096
_L = 200
_SCALE = math.sqrt(_EMB)

_NC = 2
_NS = 16
_NW = _NC * _NS

_B_TOTAL = _B * _L
_B_PER_W = _B_TOTAL // _NW
_CHUNK = 400
_N_CHUNKS = _B_PER_W // _CHUNK
_NBUF = 4


_RCHUNK = 400
_N_RCHUNKS_TOT = _VOCAB // _RCHUNK


@functools.partial(
    pl.kernel,
    out_type=jax.ShapeDtypeStruct((_VOCAB * _EMB,), jnp.float32),
    mesh=plsc.VectorSubcoreMesh(core_axis_name="c", subcore_axis_name="s"),
    scratch_types=[
        pltpu.VMEM((_RCHUNK, _EMB), jnp.float32),
        pltpu.VMEM((_RCHUNK * _EMB,), jnp.float32),
    ],
    compiler_params=pltpu.CompilerParams(use_tc_tiling_on_sc=True),
)
def _sc_scale(table_hbm, out_hbm, in_v, out_v):
    wid = lax.axis_index("s") * _NC + lax.axis_index("c")
    nk = (_N_RCHUNKS_TOT - wid + _NW - 1) // _NW

    @pl.loop(0, nk)
    def _rchunk_loop(k):
        row = (wid + k * _NW) * _RCHUNK
        pltpu.sync_copy(table_hbm.at[pl.ds(row, _RCHUNK)], in_v)

        @pl.loop(0, _RCHUNK)
        def _row_loop(i):
            for j in range(_EMB // 16):
                v = in_v[i, pl.ds(j * 16, 16)]
                out_v[pl.ds(i * _EMB + j * 16, 16)] = v * _SCALE

        pltpu.sync_copy(
            out_v, out_hbm.at[pl.ds(row * _EMB, _RCHUNK * _EMB)])


_OUT_ROWS = _L * 8 * (_B // 128) * 8


@functools.partial(
    pl.kernel,
    out_type=jax.ShapeDtypeStruct((_OUT_ROWS, 128), jnp.float32),
    mesh=plsc.VectorSubcoreMesh(core_axis_name="c", subcore_axis_name="s"),
    scratch_types=[
        pltpu.VMEM((128, _L), jnp.int32),
        pltpu.VMEM((_L * 128,), jnp.int32),
        pltpu.VMEM((_NBUF, 128, _EMB), jnp.float32),
        pltpu.VMEM((_NBUF, _EMB, 129), jnp.float32),
        pltpu.SemaphoreType.DMA((_NBUF,)),
        pltpu.SemaphoreType.DMA((_NBUF,)),
    ],
    compiler_params=pltpu.CompilerParams(
        use_tc_tiling_on_sc=False, needs_layout_passes=False),
)
def _sc_gather(table_hbm, tok_hbm, out_hbm, idx_v, idx_t, rows, trans,
               sem_g, sem_o):
    wid = lax.axis_index("s") * _NC + lax.axis_index("c")
    iota = lax.iota(jnp.int32, 16)
    zeros = jnp.zeros((16,), jnp.int32)

    pltpu.sync_copy(tok_hbm.at[pl.ds(wid * 128, 128)], idx_v)

    @pl.loop(0, _L)
    def _idx_loop(l):
        for k in range(8):
            v = plsc.load_gather(idx_v, [iota + (k * 16), zeros + l])
            idx_t[pl.ds(l * 128 + k * 16, 16)] = v

    def start_gather(b, l):
        pltpu.async_copy(
            table_hbm.at[idx_t.at[pl.ds(l * 128, 128)]],
            rows.at[b], sem_g.at[b])

    def wait_gather(b):
        pltpu.make_async_copy(
            table_hbm.at[idx_t.at[pl.ds(0, 128)]],
            rows.at[b], sem_g.at[b]).wait()

    rowidx = [iota + (c * 16) for c in range(4)]

    def transpose_scale(b):
        @pl.loop(0, 128, unroll=4)
        def _t_loop(bb):
            colv = zeros + bb
            vs = [rows[b, bb, pl.ds(c * 16, 16)] for c in range(4)]
            for c in range(4):
                plsc.store_scatter(trans.at[b], [rowidx[c], colv], vs[c])

    def start_outs(b, l):
        for et in range(8):
            row0 = ((l * 8 + et) * 32 + wid) * 8
            pltpu.async_copy(
                trans.at[b, pl.ds(et * 8, 8), pl.ds(0, 128)],
                out_hbm.at[pl.ds(row0, 8)], sem_o.at[b])

    def wait_outs(b):
        for et in range(8):
            pltpu.make_async_copy(
                trans.at[b, pl.ds(0, 8), pl.ds(0, 128)],
                out_hbm.at[pl.ds(0, 8)], sem_o.at[b]).wait()

    def group(i, first, last):
        for b in range(_NBUF):
            l = i + b
            wait_gather(b)
            if not first:
                wait_outs(b)
            transpose_scale(b)
            start_outs(b, l)
            if not last:
                start_gather(b, l + _NBUF)

    for b in range(_NBUF):
        start_gather(b, b)
    group(0, True, False)

    @pl.loop(1, _L // _NBUF - 1)
    def _main_loop(gi):
        group(gi * _NBUF, False, False)

    group(_L - _NBUF, False, True)
    for b in range(_NBUF):
        wait_outs(b)


def kernel(tokens, word_vectors):
    scaled = _sc_scale(word_vectors).reshape(_VOCAB, _EMB)
    out = _sc_gather(scaled, tokens)
    x5 = out.reshape(_L, 8, _B // 128, 8, 128)
    return x5.transpose(2, 4, 0, 1, 3).reshape(_B, _L, _EMB)


# speedup vs baseline: 1.3105x; 1.0173x over previous
import functools
import math

import jax
import jax.numpy as jnp
from jax import lax
from jax.experimental import pallas as pl
from jax.experimental.pallas import tpu as pltpu
from jax.experimental.pallas import tpu_sc as plsc

_VOCAB = 100000
_EMB = 64
_B = 4096
_L = 200
_SCALE = math.sqrt(_EMB)

_NC = 2
_NS = 16
_NW = _NC * _NS

_B_TOTAL = _B * _L
_B_PER_W = _B_TOTAL // _NW
_CHUNK = 400
_N_CHUNKS = _B_PER_W // _CHUNK
_NBUF = 4


_RCHUNK = 400
_N_RCHUNKS_TOT = _VOCAB // _RCHUNK


@functools.partial(
    pl.kernel,
    out_type=jax.ShapeDtypeStruct((_VOCAB * _EMB,), jnp.float32),
    mesh=plsc.VectorSubcoreMesh(core_axis_name="c", subcore_axis_name="s"),
    scratch_types=[
        pltpu.VMEM((_RCHUNK, _EMB), jnp.float32),
        pltpu.VMEM((_RCHUNK * _EMB,), jnp.float32),
    ],
    compiler_params=pltpu.CompilerParams(use_tc_tiling_on_sc=True),
)
def _sc_scale(table_hbm, out_hbm, in_v, out_v):
    wid = lax.axis_index("s") * _NC + lax.axis_index("c")
    nk = (_N_RCHUNKS_TOT - wid + _NW - 1) // _NW

    @pl.loop(0, nk)
    def _rchunk_loop(k):
        row = (wid + k * _NW) * _RCHUNK
        pltpu.sync_copy(table_hbm.at[pl.ds(row, _RCHUNK)], in_v)

        @pl.loop(0, _RCHUNK)
        def _row_loop(i):
            for j in range(_EMB // 16):
                v = in_v[i, pl.ds(j * 16, 16)]
                out_v[pl.ds(i * _EMB + j * 16, 16)] = v * _SCALE

        pltpu.sync_copy(
            out_v, out_hbm.at[pl.ds(row * _EMB, _RCHUNK * _EMB)])


_OUT_ROWS = _L * 8 * (_B // 128) * 8


@functools.partial(
    pl.kernel,
    out_type=jax.ShapeDtypeStruct((_L * 8, 32, 8, 128), jnp.float32),
    mesh=plsc.VectorSubcoreMesh(core_axis_name="c", subcore_axis_name="s"),
    scratch_types=[
        pltpu.VMEM((128, _L), jnp.int32),
        pltpu.VMEM((_L * 128,), jnp.int32),
        pltpu.VMEM((_NBUF, 128, _EMB), jnp.float32),
        pltpu.VMEM((_NBUF, 8, 8, 129), jnp.float32),
        pltpu.SemaphoreType.DMA((_NBUF,)),
        pltpu.SemaphoreType.DMA((_NBUF,)),
    ],
    compiler_params=pltpu.CompilerParams(
        use_tc_tiling_on_sc=False, needs_layout_passes=False),
)
def _sc_gather(table_hbm, tok_hbm, out_hbm, idx_v, idx_t, rows, trans,
               sem_g, sem_o):
    wid = lax.axis_index("s") * _NC + lax.axis_index("c")
    iota = lax.iota(jnp.int32, 16)
    zeros = jnp.zeros((16,), jnp.int32)

    pltpu.sync_copy(tok_hbm.at[pl.ds(wid * 128, 128)], idx_v)

    @pl.loop(0, _L)
    def _idx_loop(l):
        for k in range(8):
            v = plsc.load_gather(idx_v, [iota + (k * 16), zeros + l])
            idx_t[pl.ds(l * 128 + k * 16, 16)] = v

    def start_gather(b, l):
        pltpu.async_copy(
            table_hbm.at[idx_t.at[pl.ds(l * 128, 128)]],
            rows.at[b], sem_g.at[b])

    def wait_gather(b):
        pltpu.make_async_copy(
            table_hbm.at[idx_t.at[pl.ds(0, 128)]],
            rows.at[b], sem_g.at[b]).wait()

    etv = [(iota + (c * 16)) >> 3 for c in range(4)]
    e8v = [(iota + (c * 16)) & 7 for c in range(4)]

    def transpose_scale(b):
        @pl.loop(0, 128, unroll=4)
        def _t_loop(bb):
            colv = zeros + bb
            vs = [rows[b, bb, pl.ds(c * 16, 16)] for c in range(4)]
            for c in range(4):
                plsc.store_scatter(
                    trans.at[b], [etv[c], e8v[c], colv], vs[c])

    def start_outs(b, l):
        pltpu.async_copy(
            trans.at[b, pl.ds(0, 8), pl.ds(0, 8), pl.ds(0, 128)],
            out_hbm.at[pl.ds(l * 8, 8), wid], sem_o.at[b])

    def wait_outs(b):
        pltpu.make_async_copy(
            trans.at[b, pl.ds(0, 8), pl.ds(0, 8), pl.ds(0, 128)],
            out_hbm.at[pl.ds(0, 8), wid], sem_o.at[b]).wait()

    def group(i, first, last):
        for b in range(_NBUF):
            l = i + b
            wait_gather(b)
            if not first:
                wait_outs(b)
            transpose_scale(b)
            start_outs(b, l)
            if not last:
                start_gather(b, l + _NBUF)

    for b in range(_NBUF):
        start_gather(b, b)
    group(0, True, False)

    @pl.loop(1, _L // _NBUF - 1)
    def _main_loop(gi):
        group(gi * _NBUF, False, False)

    group(_L - _NBUF, False, True)
    for b in range(_NBUF):
        wait_outs(b)


def kernel(tokens, word_vectors):
    scaled = _sc_scale(word_vectors).reshape(_VOCAB, _EMB)
    out = _sc_gather(scaled, tokens)
    x5 = out.reshape(_L, 8, _B // 128, 8, 128)
    return x5.transpose(2, 4, 0, 1, 3).reshape(_B, _L, _EMB)


# speedup vs baseline: 1.4933x; 1.1395x over previous
import functools
import math

import jax
import jax.numpy as jnp
from jax import lax
from jax.experimental import pallas as pl
from jax.experimental.pallas import tpu as pltpu
from jax.experimental.pallas import tpu_sc as plsc

_VOCAB = 100000
_EMB = 64
_B = 4096
_L = 200
_SCALE = math.sqrt(_EMB)

_NC = 2
_NS = 16
_NW = _NC * _NS

_B_TOTAL = _B * _L
_B_PER_W = _B_TOTAL // _NW
_CHUNK = 400
_N_CHUNKS = _B_PER_W // _CHUNK
_NBUF = 4


_RCHUNK = 200
_N_RCHUNKS_TOT = _VOCAB // _RCHUNK


_NK_FULL = 15
_HAS8_LIM = _N_RCHUNKS_TOT - _NK_FULL * _NW


@functools.partial(
    pl.kernel,
    out_type=jax.ShapeDtypeStruct((_VOCAB * _EMB,), jnp.float32),
    mesh=plsc.VectorSubcoreMesh(core_axis_name="c", subcore_axis_name="s"),
    scratch_types=[
        pltpu.VMEM((2, _RCHUNK, _EMB), jnp.float32),
        pltpu.VMEM((2, _RCHUNK * _EMB), jnp.float32),
        pltpu.SemaphoreType.DMA((2,)),
        pltpu.SemaphoreType.DMA((2,)),
    ],
    compiler_params=pltpu.CompilerParams(use_tc_tiling_on_sc=True),
)
def _sc_scale(table_hbm, out_hbm, in_v, out_v, sem_i, sem_o):
    wid = lax.axis_index("s") * _NC + lax.axis_index("c")
    has8 = wid < _HAS8_LIM

    def start_in(b, k):
        row = (wid + k * _NW) * _RCHUNK
        pltpu.async_copy(
            table_hbm.at[pl.ds(row, _RCHUNK)], in_v.at[b], sem_i.at[b])

    def wait_in(b):
        pltpu.make_async_copy(
            table_hbm.at[pl.ds(0, _RCHUNK)], in_v.at[b], sem_i.at[b]).wait()

    def start_out(b, k):
        row = (wid + k * _NW) * _RCHUNK
        pltpu.async_copy(
            out_v.at[b], out_hbm.at[pl.ds(row * _EMB, _RCHUNK * _EMB)],
            sem_o.at[b])

    def wait_out(b):
        pltpu.make_async_copy(
            out_v.at[b], out_hbm.at[pl.ds(0, _RCHUNK * _EMB)],
            sem_o.at[b]).wait()

    def scale_body(b):
        @pl.loop(0, _RCHUNK, unroll=2)
        def _row_loop(i):
            vs = [in_v[b, i, pl.ds(j * 16, 16)] for j in range(_EMB // 16)]
            for j in range(_EMB // 16):
                out_v[b, pl.ds(i * _EMB + j * 16, 16)] = vs[j] * _SCALE

    start_in(0, 0)
    start_in(1, 1)
    for k in range(_NK_FULL):
        b = k & 1
        wait_in(b)
        if k >= 2:
            wait_out(b)
        scale_body(b)
        start_out(b, k)
        if k + 2 < _NK_FULL:
            start_in(b, k + 2)
        elif k + 2 == _NK_FULL:
            @pl.when(has8)
            def _prefetch_last():
                start_in(b, _NK_FULL)

    @pl.when(has8)
    def _last_chunk():
        wait_in(1)
        wait_out(1)
        scale_body(1)
        start_out(1, _NK_FULL)

    wait_out(0)
    wait_out(1)


_OUT_ROWS = _L * 8 * (_B // 128) * 8


@functools.partial(
    pl.kernel,
    out_type=jax.ShapeDtypeStruct((_L * 8, 32, 8, 128), jnp.float32),
    mesh=plsc.VectorSubcoreMesh(core_axis_name="c", subcore_axis_name="s"),
    scratch_types=[
        pltpu.VMEM((128, _L), jnp.int32),
        pltpu.VMEM((_L * 128,), jnp.int32),
        pltpu.VMEM((_NBUF, 128, _EMB), jnp.float32),
        pltpu.VMEM((_NBUF, 8, 8, 129), jnp.float32),
        pltpu.SemaphoreType.DMA((_NBUF,)),
        pltpu.SemaphoreType.DMA((_NBUF,)),
    ],
    compiler_params=pltpu.CompilerParams(
        use_tc_tiling_on_sc=False, needs_layout_passes=False),
)
def _sc_gather(table_hbm, tok_hbm, out_hbm, idx_v, idx_t, rows, trans,
               sem_g, sem_o):
    wid = lax.axis_index("s") * _NC + lax.axis_index("c")
    iota = lax.iota(jnp.int32, 16)
    zeros = jnp.zeros((16,), jnp.int32)

    pltpu.sync_copy(tok_hbm.at[pl.ds(wid * 128, 128)], idx_v)

    @pl.loop(0, _L)
    def _idx_loop(l):
        for k in range(8):
            v = plsc.load_gather(idx_v, [iota + (k * 16), zeros + l])
            idx_t[pl.ds(l * 128 + k * 16, 16)] = v

    def start_gather(b, l):
        pltpu.async_copy(
            table_hbm.at[idx_t.at[pl.ds(l * 128, 128)]],
            rows.at[b], sem_g.at[b])

    def wait_gather(b):
        pltpu.make_async_copy(
            table_hbm.at[idx_t.at[pl.ds(0, 128)]],
            rows.at[b], sem_g.at[b]).wait()

    etv = [(iota + (c * 16)) >> 3 for c in range(4)]
    e8v = [(iota + (c * 16)) & 7 for c in range(4)]

    def transpose_scale(b):
        @pl.loop(0, 128, unroll=4)
        def _t_loop(bb):
            colv = zeros + bb
            vs = [rows[b, bb, pl.ds(c * 16, 16)] for c in range(4)]
            for c in range(4):
                plsc.store_scatter(
                    trans.at[b], [etv[c], e8v[c], colv], vs[c])

    def start_outs(b, l):
        pltpu.async_copy(
            trans.at[b, pl.ds(0, 8), pl.ds(0, 8), pl.ds(0, 128)],
            out_hbm.at[pl.ds(l * 8, 8), wid], sem_o.at[b])

    def wait_outs(b):
        pltpu.make_async_copy(
            trans.at[b, pl.ds(0, 8), pl.ds(0, 8), pl.ds(0, 128)],
            out_hbm.at[pl.ds(0, 8), wid], sem_o.at[b]).wait()

    def group(i, first, last):
        for b in range(_NBUF):
            l = i + b
            wait_gather(b)
            if not first:
                wait_outs(b)
            transpose_scale(b)
            start_outs(b, l)
            if not last:
                start_gather(b, l + _NBUF)

    for b in range(_NBUF):
        start_gather(b, b)
    group(0, True, False)

    @pl.loop(1, _L // _NBUF - 1)
    def _main_loop(gi):
        group(gi * _NBUF, False, False)

    group(_L - _NBUF, False, True)
    for b in range(_NBUF):
        wait_outs(b)


def kernel(tokens, word_vectors):
    scaled = _sc_scale(word_vectors).reshape(_VOCAB, _EMB)
    out = _sc_gather(scaled, tokens)
    x5 = out.reshape(_L, 8, _B // 128, 8, 128)
    return x5.transpose(2, 4, 0, 1, 3).reshape(_B, _L, _EMB)


# speedup vs baseline: 1.5574x; 1.0429x over previous
import functools
import math

import jax
import jax.numpy as jnp
from jax import lax
from jax.experimental import pallas as pl
from jax.experimental.pallas import tpu as pltpu
from jax.experimental.pallas import tpu_sc as plsc

_VOCAB = 100000
_EMB = 64
_B = 4096
_L = 200
_SCALE = math.sqrt(_EMB)

_NC = 2
_NS = 16
_NW = _NC * _NS

_B_TOTAL = _B * _L
_B_PER_W = _B_TOTAL // _NW
_CHUNK = 400
_N_CHUNKS = _B_PER_W // _CHUNK
_NBUF = 4


_RCHUNK = 200
_N_RCHUNKS_TOT = _VOCAB // _RCHUNK


_NK_FULL = 15
_HAS8_LIM = _N_RCHUNKS_TOT - _NK_FULL * _NW


@functools.partial(
    pl.kernel,
    out_type=jax.ShapeDtypeStruct((_VOCAB * _EMB,), jnp.float32),
    mesh=plsc.VectorSubcoreMesh(core_axis_name="c", subcore_axis_name="s"),
    scratch_types=[
        pltpu.VMEM((2, _RCHUNK, _EMB), jnp.float32),
        pltpu.VMEM((2, _RCHUNK * _EMB), jnp.float32),
        pltpu.SemaphoreType.DMA((2,)),
        pltpu.SemaphoreType.DMA((2,)),
    ],
    compiler_params=pltpu.CompilerParams(use_tc_tiling_on_sc=True),
)
def _sc_scale(table_hbm, out_hbm, in_v, out_v, sem_i, sem_o):
    wid = lax.axis_index("s") * _NC + lax.axis_index("c")
    has8 = wid < _HAS8_LIM

    def start_in(b, k):
        row = (wid + k * _NW) * _RCHUNK
        pltpu.async_copy(
            table_hbm.at[pl.ds(row, _RCHUNK)], in_v.at[b], sem_i.at[b])

    def wait_in(b):
        pltpu.make_async_copy(
            table_hbm.at[pl.ds(0, _RCHUNK)], in_v.at[b], sem_i.at[b]).wait()

    def start_out(b, k):
        row = (wid + k * _NW) * _RCHUNK
        pltpu.async_copy(
            out_v.at[b], out_hbm.at[pl.ds(row * _EMB, _RCHUNK * _EMB)],
            sem_o.at[b])

    def wait_out(b):
        pltpu.make_async_copy(
            out_v.at[b], out_hbm.at[pl.ds(0, _RCHUNK * _EMB)],
            sem_o.at[b]).wait()

    def scale_body(b):
        @pl.loop(0, _RCHUNK, unroll=2)
        def _row_loop(i):
            vs = [in_v[b, i, pl.ds(j * 16, 16)] for j in range(_EMB // 16)]
            for j in range(_EMB // 16):
                out_v[b, pl.ds(i * _EMB + j * 16, 16)] = vs[j] * _SCALE

    start_in(0, 0)
    start_in(1, 1)
    for k in range(_NK_FULL):
        b = k & 1
        wait_in(b)
        if k >= 2:
            wait_out(b)
        scale_body(b)
        start_out(b, k)
        if k + 2 < _NK_FULL:
            start_in(b, k + 2)
        elif k + 2 == _NK_FULL:
            @pl.when(has8)
            def _prefetch_last():
                start_in(b, _NK_FULL)

    @pl.when(has8)
    def _last_chunk():
        wait_in(1)
        wait_out(1)
        scale_body(1)
        start_out(1, _NK_FULL)

    wait_out(0)
    wait_out(1)


_OUT_ROWS = _L * 8 * (_B // 128) * 8


@functools.partial(
    pl.kernel,
    out_type=jax.ShapeDtypeStruct((_L * 8, 32, 8, 128), jnp.float32),
    mesh=plsc.VectorSubcoreMesh(core_axis_name="c", subcore_axis_name="s"),
    scratch_types=[
        pltpu.VMEM((128, _L), jnp.int32),
        pltpu.VMEM((_L * 128,), jnp.int32),
        pltpu.VMEM((_NBUF, 128, _EMB), jnp.float32),
        pltpu.VMEM((_NBUF, 8, 8, 129), jnp.float32),
        pltpu.SemaphoreType.DMA((_NBUF,)),
        pltpu.SemaphoreType.DMA((_NBUF,)),
    ],
    compiler_params=pltpu.CompilerParams(
        use_tc_tiling_on_sc=False, needs_layout_passes=False),
)
def _sc_gather(table_hbm, tok_hbm, out_hbm, idx_v, idx_t, rows, trans,
               sem_g, sem_o):
    wid = lax.axis_index("s") * _NC + lax.axis_index("c")
    iota = lax.iota(jnp.int32, 16)
    zeros = jnp.zeros((16,), jnp.int32)

    pltpu.sync_copy(tok_hbm.at[pl.ds(wid * 128, 128)], idx_v)

    @pl.loop(0, _L)
    def _idx_loop(l):
        for k in range(8):
            v = plsc.load_gather(idx_v, [iota + (k * 16), zeros + l])
            idx_t[pl.ds(l * 128 + k * 16, 16)] = v

    def start_gather(b, l):
        pltpu.async_copy(
            table_hbm.at[idx_t.at[pl.ds(l * 128, 128)]],
            rows.at[b], sem_g.at[b])

    def wait_gather(b):
        pltpu.make_async_copy(
            table_hbm.at[idx_t.at[pl.ds(0, 128)]],
            rows.at[b], sem_g.at[b]).wait()

    etv = [(iota + (c * 16)) >> 3 for c in range(4)]
    e8v = [(iota + (c * 16)) & 7 for c in range(4)]

    def transpose_scale(b):
        @pl.loop(0, 64, unroll=2)
        def _t_loop(ii):
            bb0 = ii * 2
            colv0 = zeros + bb0
            colv1 = zeros + (bb0 + 1)
            vs0 = [rows[b, bb0, pl.ds(c * 16, 16)] for c in range(4)]
            vs1 = [rows[b, bb0 + 1, pl.ds(c * 16, 16)] for c in range(4)]
            for c in range(4):
                plsc.store_scatter(
                    trans.at[b], [etv[c], e8v[c], colv0], vs0[c])
            for c in range(4):
                plsc.store_scatter(
                    trans.at[b], [etv[c], e8v[c], colv1], vs1[c])

    def start_outs(b, l):
        pltpu.async_copy(
            trans.at[b, pl.ds(0, 8), pl.ds(0, 8), pl.ds(0, 128)],
            out_hbm.at[pl.ds(l * 8, 8), wid], sem_o.at[b])

    def wait_outs(b):
        pltpu.make_async_copy(
            trans.at[b, pl.ds(0, 8), pl.ds(0, 8), pl.ds(0, 128)],
            out_hbm.at[pl.ds(0, 8), wid], sem_o.at[b]).wait()

    def group(i, first, last):
        for b in range(_NBUF):
            l = i + b
            wait_gather(b)
            if not first:
                wait_outs(b)
            transpose_scale(b)
            start_outs(b, l)
            if not last:
                start_gather(b, l + _NBUF)

    for b in range(_NBUF):
        start_gather(b, b)
    group(0, True, False)

    @pl.loop(1, _L // _NBUF - 1)
    def _main_loop(gi):
        group(gi * _NBUF, False, False)

    group(_L - _NBUF, False, True)
    for b in range(_NBUF):
        wait_outs(b)


def kernel(tokens, word_vectors):
    scaled = _sc_scale(word_vectors).reshape(_VOCAB, _EMB)
    out = _sc_gather(scaled, tokens)
    x5 = out.reshape(_L, 8, _B // 128, 8, 128)
    return x5.transpose(2, 4, 0, 1, 3).reshape(_B, _L, _EMB)


# speedup vs baseline: 1.5589x; 1.0010x over previous
import functools
import math

import jax
import jax.numpy as jnp
from jax import lax
from jax.experimental import pallas as pl
from jax.experimental.pallas import tpu as pltpu
from jax.experimental.pallas import tpu_sc as plsc

_VOCAB = 100000
_EMB = 64
_B = 4096
_L = 200
_SCALE = math.sqrt(_EMB)

_NC = 2
_NS = 16
_NW = _NC * _NS

_NBUF = 4

_RCHUNK = 200
_N_RCHUNKS_TOT = _VOCAB // _RCHUNK


_NK_FULL = 15
_HAS8_LIM = _N_RCHUNKS_TOT - _NK_FULL * _NW


@functools.partial(
    pl.kernel,
    out_type=jax.ShapeDtypeStruct((_VOCAB * _EMB,), jnp.float32),
    mesh=plsc.VectorSubcoreMesh(core_axis_name="c", subcore_axis_name="s"),
    scratch_types=[
        pltpu.VMEM((2, _RCHUNK, _EMB), jnp.float32),
        pltpu.VMEM((2, _RCHUNK * _EMB), jnp.float32),
        pltpu.SemaphoreType.DMA((2,)),
        pltpu.SemaphoreType.DMA((2,)),
    ],
    compiler_params=pltpu.CompilerParams(use_tc_tiling_on_sc=True),
)
def _sc_scale(table_hbm, out_hbm, in_v, out_v, sem_i, sem_o):
    wid = lax.axis_index("s") * _NC + lax.axis_index("c")
    has8 = wid < _HAS8_LIM

    def start_in(b, k):
        row = (wid + k * _NW) * _RCHUNK
        pltpu.async_copy(
            table_hbm.at[pl.ds(row, _RCHUNK)], in_v.at[b], sem_i.at[b])

    def wait_in(b):
        pltpu.make_async_copy(
            table_hbm.at[pl.ds(0, _RCHUNK)], in_v.at[b], sem_i.at[b]).wait()

    def start_out(b, k):
        row = (wid + k * _NW) * _RCHUNK
        pltpu.async_copy(
            out_v.at[b], out_hbm.at[pl.ds(row * _EMB, _RCHUNK * _EMB)],
            sem_o.at[b])

    def wait_out(b):
        pltpu.make_async_copy(
            out_v.at[b], out_hbm.at[pl.ds(0, _RCHUNK * _EMB)],
            sem_o.at[b]).wait()

    def scale_body(b):
        @pl.loop(0, _RCHUNK, unroll=2)
        def _row_loop(i):
            vs = [in_v[b, i, pl.ds(j * 16, 16)] for j in range(_EMB // 16)]
            for j in range(_EMB // 16):
                out_v[b, pl.ds(i * _EMB + j * 16, 16)] = vs[j] * _SCALE

    start_in(0, 0)
    start_in(1, 1)
    for k in range(_NK_FULL):
        b = k & 1
        wait_in(b)
        if k >= 2:
            wait_out(b)
        scale_body(b)
        start_out(b, k)
        if k + 2 < _NK_FULL:
            start_in(b, k + 2)
        elif k + 2 == _NK_FULL:
            @pl.when(has8)
            def _prefetch_last():
                start_in(b, _NK_FULL)

    @pl.when(has8)
    def _last_chunk():
        wait_in(1)
        wait_out(1)
        scale_body(1)
        start_out(1, _NK_FULL)

    wait_out(0)
    wait_out(1)


@functools.partial(
    pl.kernel,
    out_type=jax.ShapeDtypeStruct((_L * 8, 32, 8, 128), jnp.float32),
    mesh=plsc.VectorSubcoreMesh(core_axis_name="c", subcore_axis_name="s"),
    scratch_types=[
        pltpu.VMEM((128, _L), jnp.int32),
        pltpu.VMEM((_L * 128,), jnp.int32),
        pltpu.VMEM((_NBUF, 128, _EMB), jnp.float32),
        pltpu.VMEM((_NBUF, 8, 8, 129), jnp.float32),
        pltpu.SemaphoreType.DMA((_NBUF,)),
        pltpu.SemaphoreType.DMA((_NBUF,)),
    ],
    compiler_params=pltpu.CompilerParams(
        use_tc_tiling_on_sc=False, needs_layout_passes=False),
)
def _sc_gather(table_hbm, tok_hbm, out_hbm, idx_v, idx_t, rows, trans,
               sem_g, sem_o):
    wid = lax.axis_index("s") * _NC + lax.axis_index("c")
    iota = lax.iota(jnp.int32, 16)
    zeros = jnp.zeros((16,), jnp.int32)

    pltpu.sync_copy(tok_hbm.at[pl.ds(wid * 128, 128)], idx_v)

    @pl.loop(0, _L)
    def _idx_loop(l):
        for k in range(8):
            v = plsc.load_gather(idx_v, [iota + (k * 16), zeros + l])
            idx_t[pl.ds(l * 128 + k * 16, 16)] = v

    def start_gather(b, l):
        pltpu.async_copy(
            table_hbm.at[idx_t.at[pl.ds(l * 128, 128)]],
            rows.at[b], sem_g.at[b])

    def wait_gather(b):
        pltpu.make_async_copy(
            table_hbm.at[idx_t.at[pl.ds(0, 128)]],
            rows.at[b], sem_g.at[b]).wait()

    etv = [(iota + (c * 16)) >> 3 for c in range(4)]
    e8v = [(iota + (c * 16)) & 7 for c in range(4)]

    def transpose_scale(b):
        @pl.loop(0, 64, unroll=2)
        def _t_loop(ii):
            bb0 = ii * 2
            colv0 = zeros + bb0
            colv1 = zeros + (bb0 + 1)
            vs0 = [rows[b, bb0, pl.ds(c * 16, 16)] for c in range(4)]
            vs1 = [rows[b, bb0 + 1, pl.ds(c * 16, 16)] for c in range(4)]
            for c in range(4):
                plsc.store_scatter(
                    trans.at[b], [etv[c], e8v[c], colv0], vs0[c])
            for c in range(4):
                plsc.store_scatter(
                    trans.at[b], [etv[c], e8v[c], colv1], vs1[c])

    def start_outs(b, l):
        pltpu.async_copy(
            trans.at[b, pl.ds(0, 8), pl.ds(0, 8), pl.ds(0, 128)],
            out_hbm.at[pl.ds(l * 8, 8), wid], sem_o.at[b])

    def wait_outs(b):
        pltpu.make_async_copy(
            trans.at[b, pl.ds(0, 8), pl.ds(0, 8), pl.ds(0, 128)],
            out_hbm.at[pl.ds(0, 8), wid], sem_o.at[b]).wait()

    def group(i, first, last):
        for b in range(_NBUF):
            l = i + b
            wait_gather(b)
            if not first:
                wait_outs(b)
            transpose_scale(b)
            start_outs(b, l)
            if not last:
                start_gather(b, l + _NBUF)

    for b in range(_NBUF):
        start_gather(b, b)
    group(0, True, False)

    @pl.loop(1, _L // _NBUF - 1)
    def _main_loop(gi):
        group(gi * _NBUF, False, False)

    group(_L - _NBUF, False, True)
    for b in range(_NBUF):
        wait_outs(b)


def kernel(tokens, word_vectors):
    scaled = _sc_scale(word_vectors).reshape(_VOCAB, _EMB)
    out = _sc_gather(scaled, tokens)
    x5 = out.reshape(_L, 8, _B // 128, 8, 128)
    return x5.transpose(2, 4, 0, 1, 3).reshape(_B, _L, _EMB)
